# Initial kernel scaffold; baseline (speedup 1.0000x reference)
import jax
import jax.numpy as jnp
from jax.experimental import pallas as pl


def kernel(t, gamma):
    raise NotImplementedError("write your pallas kernel here")


# speedup vs baseline: 4.5121x; 4.5121x over previous
import jax
import jax.numpy as jnp
from jax import lax
from jax.experimental import pallas as pl
from jax.experimental.pallas import tpu as pltpu
from jax.experimental.pallas import tpu_sc as plsc

_TIMESTEPS = 1000
_N = 16384
_NC = 2
_NS = 16
_L = 16
_NW = _NC * _NS
_CHUNK = _N // _NW
_TABLE_PAD = 1024


def _lookup_body(t_hbm, gamma_hbm, out_hbm, t_v, gamma_v, out_v):
    wid = lax.axis_index("s") * _NC + lax.axis_index("c")
    base = wid * _CHUNK
    pltpu.sync_copy(t_hbm.at[pl.ds(base, _CHUNK)], t_v)
    pltpu.sync_copy(gamma_hbm, gamma_v)
    for i in range(_CHUNK // _L):
        x = t_v[pl.ds(i * _L, _L)] * float(_TIMESTEPS)
        idx = (x + 0.5).astype(jnp.int32)
        tie = (idx.astype(jnp.float32) - x) == 0.5
        odd = lax.bitwise_and(idx, 1) == 1
        idx = jnp.where(jnp.logical_and(tie, odd), idx - 1, idx)
        out_v[pl.ds(i * _L, _L)] = plsc.load_gather(gamma_v, [idx])
    pltpu.sync_copy(out_v, out_hbm.at[pl.ds(base, _CHUNK)])


@jax.jit
def kernel(t, gamma):
    pad = _TABLE_PAD - gamma.shape[0]
    gamma_pad = jnp.concatenate([gamma, jnp.zeros((pad,), gamma.dtype)])
    run = pl.kernel(
        _lookup_body,
        out_type=jax.ShapeDtypeStruct((_N,), jnp.float32),
        mesh=plsc.VectorSubcoreMesh(core_axis_name="c", subcore_axis_name="s"),
        scratch_types=[
            pltpu.VMEM((_CHUNK,), jnp.float32),
            pltpu.VMEM((_TABLE_PAD,), jnp.float32),
            pltpu.VMEM((_CHUNK,), jnp.float32),
        ],
        compiler_params=pltpu.CompilerParams(needs_layout_passes=False),
    )
    return run(t, gamma_pad)


# speedup vs baseline: 4.5403x; 1.0063x over previous
import jax
import jax.numpy as jnp
from jax import lax
from jax.experimental import pallas as pl
from jax.experimental.pallas import tpu as pltpu
from jax.experimental.pallas import tpu_sc as plsc

_TIMESTEPS = 1000
_N = 16384
_NC = 2
_NS = 16
_L = 16
_NW = _NC * _NS
_CHUNK = _N // _NW
_TABLE_PAD = 1024


def _lookup_body(t_hbm, gamma_hbm, out_hbm, t_v, gamma_v, out_v, sem_t, sem_g):
    wid = lax.axis_index("s") * _NC + lax.axis_index("c")
    base = wid * _CHUNK
    cp_t = pltpu.async_copy(t_hbm.at[pl.ds(base, _CHUNK)], t_v, sem_t)
    cp_g = pltpu.async_copy(gamma_hbm, gamma_v, sem_g)
    cp_t.wait()
    cp_g.wait()
    for i in range(_CHUNK // _L):
        x = t_v[pl.ds(i * _L, _L)] * float(_TIMESTEPS)
        idx = (x + 0.5).astype(jnp.int32)
        tie = (idx.astype(jnp.float32) - x) == 0.5
        odd = lax.bitwise_and(idx, 1) == 1
        idx = jnp.where(jnp.logical_and(tie, odd), idx - 1, idx)
        out_v[pl.ds(i * _L, _L)] = plsc.load_gather(gamma_v, [idx])
    pltpu.sync_copy(out_v, out_hbm.at[pl.ds(base, _CHUNK)])


@jax.jit
def kernel(t, gamma):
    run = pl.kernel(
        _lookup_body,
        out_type=jax.ShapeDtypeStruct((_N,), jnp.float32),
        mesh=plsc.VectorSubcoreMesh(core_axis_name="c", subcore_axis_name="s"),
        scratch_types=[
            pltpu.VMEM((_CHUNK,), jnp.float32),
            pltpu.VMEM((1001,), jnp.float32),
            pltpu.VMEM((_CHUNK,), jnp.float32),
            pltpu.SemaphoreType.DMA,
            pltpu.SemaphoreType.DMA,
        ],
        compiler_params=pltpu.CompilerParams(needs_layout_passes=False),
    )
    return run(t, gamma)


# speedup vs baseline: 4.5550x; 1.0032x over previous
import jax
import jax.numpy as jnp
from jax import lax
from jax.experimental import pallas as pl
from jax.experimental.pallas import tpu as pltpu
from jax.experimental.pallas import tpu_sc as plsc

_TIMESTEPS = 1000
_N = 16384
_NC = 2
_NS = 16
_L = 16
_NW = _NC * _NS
_CHUNK = _N // _NW
_TABLE_PAD = 1024


def _lookup_body(t_hbm, gamma_hbm, out_hbm, t_v, gamma_v, out_v, sem_t, sem_g):
    wid = lax.axis_index("s") * _NC + lax.axis_index("c")
    base = wid * _CHUNK
    cp_t = pltpu.async_copy(t_hbm.at[pl.ds(base, _CHUNK)], t_v, sem_t)
    cp_g = pltpu.async_copy(gamma_hbm, gamma_v, sem_g)
    cp_t.wait()
    cp_g.wait()
    for i in range(_CHUNK // _L):
        x = t_v[pl.ds(i * _L, _L)] * float(_TIMESTEPS)
        idx = (x + 0.5).astype(jnp.int32)
        tie = (idx.astype(jnp.float32) - x) == 0.5
        odd = lax.bitwise_and(idx, 1) == 1
        idx = jnp.where(jnp.logical_and(tie, odd), idx - 1, idx)
        out_v[pl.ds(i * _L, _L)] = plsc.load_gather(gamma_v, [idx])
    pltpu.sync_copy(out_v, out_hbm.at[pl.ds(base, _CHUNK)])


@jax.jit
def kernel(t, gamma):
    run = pl.kernel(
        _lookup_body,
        out_type=jax.ShapeDtypeStruct((_N,), jnp.float32),
        mesh=plsc.VectorSubcoreMesh(core_axis_name="c", subcore_axis_name="s"),
        scratch_types=[
            pltpu.VMEM((_CHUNK,), jnp.float32),
            pltpu.VMEM((1001,), jnp.float32),
            pltpu.VMEM((_CHUNK,), jnp.float32),
            pltpu.SemaphoreType.DMA,
            pltpu.SemaphoreType.DMA,
        ],
        compiler_params=pltpu.CompilerParams(
            needs_layout_passes=False,
            skip_device_barrier=True,
            disable_bounds_checks=True,
            disable_semaphore_checks=True,
        ),
    )
    return run(t, gamma)


# speedup vs baseline: 4.6646x; 1.0240x over previous
import jax
import jax.numpy as jnp
from jax import lax
from jax.experimental import pallas as pl
from jax.experimental.pallas import tpu as pltpu
from jax.experimental.pallas import tpu_sc as plsc

_TIMESTEPS = 1000
_N = 16384
_NC = 2
_NS = 16
_L = 16
_NW = _NC * _NS
_CHUNK = _N // _NW
_TABLE_PAD = 1024


def _lookup_body(t_hbm, gamma_hbm, out_hbm, t_v, gamma_v, out_v, sem_t, sem_g):
    wid = lax.axis_index("s") * _NC + lax.axis_index("c")
    base = wid * _CHUNK
    cp_t = pltpu.async_copy(t_hbm.at[pl.ds(base, _CHUNK)], t_v, sem_t)
    cp_g = pltpu.async_copy(gamma_hbm, gamma_v, sem_g)
    cp_t.wait()
    cp_g.wait()
    @pl.loop(0, _CHUNK, step=_L, unroll=4)
    def _(off):
        x = t_v[pl.ds(off, _L)] * float(_TIMESTEPS)
        idx = (x + 0.5).astype(jnp.int32)
        tie = (idx.astype(jnp.float32) - x) == 0.5
        odd = lax.bitwise_and(idx, 1) == 1
        idx = jnp.where(jnp.logical_and(tie, odd), idx - 1, idx)
        out_v[pl.ds(off, _L)] = plsc.load_gather(gamma_v, [idx])
    pltpu.sync_copy(out_v, out_hbm.at[pl.ds(base, _CHUNK)])


@jax.jit
def kernel(t, gamma):
    run = pl.kernel(
        _lookup_body,
        out_type=jax.ShapeDtypeStruct((_N,), jnp.float32),
        mesh=plsc.VectorSubcoreMesh(core_axis_name="c", subcore_axis_name="s"),
        scratch_types=[
            pltpu.VMEM((_CHUNK,), jnp.float32),
            pltpu.VMEM((1001,), jnp.float32),
            pltpu.VMEM((_CHUNK,), jnp.float32),
            pltpu.SemaphoreType.DMA,
            pltpu.SemaphoreType.DMA,
        ],
        compiler_params=pltpu.CompilerParams(
            needs_layout_passes=False,
            skip_device_barrier=True,
            disable_bounds_checks=True,
            disable_semaphore_checks=True,
        ),
    )
    return run(t, gamma)
